# baseline (speedup 1.0000x reference)
import functools

import jax
import jax.numpy as jnp
from jax import lax
from jax.experimental import pallas as pl
from jax.experimental.pallas import tpu as pltpu
from jax.experimental.pallas import tpu_sc as plsc

NUM_FRAME = 8
TOPK = 64
TOP_ATTN = 6
P = 196
D = 768
NUM_LAYERS = 12
NUM_HEADS = 12
SEQ = P + 1
W = 256

SC_CORES = 2
SC_SUBCORES = 16

NL = NUM_LAYERS - TOP_ATTN
HG = 2
HPG = NUM_HEADS // HG
NUNITS = NL * HG
NK = W // 16

_OFFS = [k * 16 for k in range(SEQ // 16)] + [SEQ - 16]


@functools.lru_cache(maxsize=None)
def _make_sc_scores(batch):
    assert batch == SC_CORES
    mesh = plsc.VectorSubcoreMesh(core_axis_name="c", subcore_axis_name="s")

    @functools.partial(
        pl.kernel,
        mesh=mesh,
        compiler_params=pltpu.CompilerParams(use_tc_tiling_on_sc=True),
        out_type=jax.ShapeDtypeStruct((batch * NUM_FRAME * W,), jnp.float32),
        scratch_types=[
            pltpu.VMEM((HPG, NUM_FRAME, SEQ), jnp.float32),
            pltpu.VMEM((NUM_FRAME, W), jnp.float32),
            pltpu.VMEM((NUNITS, NUM_FRAME, W), jnp.float32),
            pltpu.VMEM((W,), jnp.float32),
            pltpu.VMEM_SHARED((NUNITS, NUM_FRAME, W), jnp.float32),
            pltpu.SemaphoreType.DMA,
        ],
    )
    def sc_scores(attn_hbm, scores_hbm, buf, acc, cmb, srow, parts_sh, sem):
        c = lax.axis_index("c")
        s = lax.axis_index("s")
        b = c

        @pl.when(s < NUNITS)
        def _fetch_reduce():
            l = TOP_ATTN + s // HG
            hg = s % HG
            pltpu.async_copy(
                attn_hbm.at[b, l, pl.ds(hg * HPG, HPG), 0, :, :],
                buf,
                sem,
            ).wait()
            for t in range(NUM_FRAME):
                for o in _OFFS:
                    v = buf[0, t, pl.ds(o, 16)]
                    for j in range(1, HPG):
                        v = v + buf[j, t, pl.ds(o, 16)]
                    acc[t, pl.ds(o, 16)] = v
            pltpu.sync_copy(acc, parts_sh.at[s])

        plsc.subcore_barrier()

        @pl.when(s < NUM_FRAME)
        def _combine():
            pltpu.sync_copy(parts_sh, cmb)
            t = s
            r = b * NUM_FRAME + t
            for k in range(SEQ // 16 + 1):
                o = min(k * 16, W - 16)
                v = cmb[0, t, pl.ds(o, 16)]
                for u in range(1, NUNITS):
                    v = v + cmb[u, t, pl.ds(o, 16)]
                srow[pl.ds(o, 16)] = v
            pltpu.sync_copy(srow, scores_hbm.at[pl.ds(r * W, W)])

    return sc_scores


@functools.lru_cache(maxsize=None)
def _make_sc_topk_gather(batch):
    rows = batch * NUM_FRAME
    assert rows <= SC_CORES * SC_SUBCORES
    n_rows = rows * TOPK
    mesh = plsc.VectorSubcoreMesh(core_axis_name="c", subcore_axis_name="s")

    @functools.partial(
        pl.kernel,
        mesh=mesh,
        out_type=[
            jax.ShapeDtypeStruct((n_rows, D), jnp.float32),
            jax.ShapeDtypeStruct((n_rows,), jnp.int32),
        ],
        scratch_types=[
            pltpu.VMEM((W,), jnp.float32),
            pltpu.VMEM((32,), jnp.float32),
            pltpu.VMEM((32,), jnp.int32),
            pltpu.VMEM((TOPK,), jnp.int32),
            pltpu.VMEM((TOPK, D), jnp.float32),
            pltpu.SemaphoreType.DMA,
        ],
    )
    def sc_topk_gather(scores_hbm, table_hbm, out_hbm, idx_hbm,
                       srow, tmpf, tmpi, i1d, rows_v, sem):
        wid = lax.axis_index("s") * SC_CORES + lax.axis_index("c")

        @pl.when(wid < rows)
        def _():
            r = wid
            pltpu.sync_copy(scores_hbm.at[pl.ds(r * W, W)], srow)
            lane16 = lax.iota(jnp.int32, 16)
            ninf = jnp.float32(-jnp.inf)
            tmpf[pl.ds(16, 16)] = jnp.full((16,), ninf)
            tmpi[pl.ds(16, 16)] = jnp.full((16,), jnp.int32(2**30))

            def xlane_max(v):
                for sh in (8, 4, 2, 1):
                    tmpf[pl.ds(0, 16)] = v
                    v = jnp.maximum(v, tmpf[pl.ds(sh, 16)])
                return v[0]

            def xlane_min(v):
                for sh in (8, 4, 2, 1):
                    tmpi[pl.ds(0, 16)] = v
                    v = jnp.minimum(v, tmpi[pl.ds(sh, 16)])
                return v[0]

            sv = []
            for k in range(NK):
                v = srow[pl.ds(k * 16, 16)]
                if k == 0:
                    v = jnp.where(lane16 == 0, ninf, v)
                if k * 16 >= SEQ:
                    v = jnp.full((16,), ninf)
                elif (k + 1) * 16 > SEQ:
                    v = jnp.where(lane16 < SEQ - k * 16, v, ninf)
                sv.append(v)

            def step(k, carry):
                sv = list(carry[:NK])
                iv = list(carry[NK:])
                m = sv[0]
                for g in range(1, NK):
                    m = jnp.maximum(m, sv[g])
                m = xlane_max(m)
                cand = None
                big = jnp.int32(2**30)
                for g in range(NK):
                    cg = jnp.where(sv[g] == m, lane16 + g * 16, big)
                    cand = cg if cand is None else jnp.minimum(cand, cg)
                glane = xlane_min(cand)
                patch = glane - 1
                for g in range(NK):
                    sv[g] = jnp.where(glane == lane16 + g * 16, ninf, sv[g])
                for g in range(TOPK // 16):
                    iv[g] = jnp.where(lane16 + g * 16 == k, patch, iv[g])
                return tuple(sv) + tuple(iv)

            iv0 = [jnp.zeros((16,), jnp.int32) for _ in range(TOPK // 16)]
            carry = lax.fori_loop(0, TOPK, step, tuple(sv) + tuple(iv0))
            iv = carry[NK:]

            for g in range(TOPK // 16):
                i1d[pl.ds(g * 16, 16)] = iv[g]
            pltpu.sync_copy(i1d, idx_hbm.at[pl.ds(r * TOPK, TOPK)])

            gbase = r * P
            handles = [
                pltpu.async_copy(
                    table_hbm.at[iv[g] + gbase],
                    rows_v.at[pl.ds(g * 16, 16)],
                    sem,
                )
                for g in range(TOPK // 16)
            ]
            for h in handles:
                h.wait()
            pltpu.sync_copy(rows_v, out_hbm.at[pl.ds(r * TOPK, TOPK)])

    return sc_topk_gather


def kernel(tokens, attn_maps):
    B = tokens.shape[0]

    am_t = jnp.transpose(attn_maps, (0, 2, 3, 4, 1, 5))
    scores = _make_sc_scores(B)(am_t)
    out, idx = _make_sc_topk_gather(B)(
        scores, tokens.reshape(B * NUM_FRAME * P, D)
    )

    return (
        out.reshape(B, NUM_FRAME * TOPK, D),
        idx.reshape(B, NUM_FRAME, TOPK),
    )


# speedup vs baseline: 1.2147x; 1.2147x over previous
import functools

import jax
import jax.numpy as jnp
from jax import lax
from jax.experimental import pallas as pl
from jax.experimental.pallas import tpu as pltpu
from jax.experimental.pallas import tpu_sc as plsc

NUM_FRAME = 8
TOPK = 64
TOP_ATTN = 6
P = 196
D = 768
NUM_LAYERS = 12
NUM_HEADS = 12
SEQ = P + 1
W = 208

SC_CORES = 2
SC_SUBCORES = 16
SC_WORKERS = SC_CORES * SC_SUBCORES

NL = NUM_LAYERS - TOP_ATTN
HG = 2
HPG = NUM_HEADS // HG

_OFFS = [k * 16 for k in range(SEQ // 16)] + [SEQ - 16]


@functools.lru_cache(maxsize=None)
def _make_sc_scores(batch):
    n_units = batch * NL * HG
    assert n_units <= SC_WORKERS
    mesh = plsc.VectorSubcoreMesh(core_axis_name="c", subcore_axis_name="s")

    @functools.partial(
        pl.kernel,
        mesh=mesh,
        compiler_params=pltpu.CompilerParams(use_tc_tiling_on_sc=True),
        out_type=jax.ShapeDtypeStruct((n_units, NUM_FRAME, W), jnp.float32),
        scratch_types=[
            pltpu.VMEM((HPG, NUM_FRAME, SEQ), jnp.float32),
            pltpu.VMEM((NUM_FRAME, W), jnp.float32),
            pltpu.SemaphoreType.DMA,
        ],
    )
    def sc_scores(attn_hbm, out_hbm, buf, acc, sem):
        wid = lax.axis_index("s") * SC_CORES + lax.axis_index("c")

        @pl.when(wid < n_units)
        def _():
            b = wid // (NL * HG)
            rem = wid % (NL * HG)
            l = TOP_ATTN + rem // HG
            hg = rem % HG
            pltpu.async_copy(
                attn_hbm.at[b, l, pl.ds(hg * HPG, HPG), 0, :, :],
                buf,
                sem,
            ).wait()
            for t in range(NUM_FRAME):
                for o in _OFFS:
                    s = buf[0, t, pl.ds(o, 16)]
                    for j in range(1, HPG):
                        s = s + buf[j, t, pl.ds(o, 16)]
                    acc[t, pl.ds(o, 16)] = s
            pltpu.sync_copy(acc, out_hbm.at[wid])

    return sc_scores


def _topk_body(s_ref, idx_ref, gidx_ref, *, batch):
    rows = batch * NUM_FRAME
    s = jnp.sum(s_ref[...], axis=1).reshape(rows, W)

    lane = lax.broadcasted_iota(jnp.int32, (rows, W), 1)
    valid = (lane >= 1) & (lane < SEQ)
    s = jnp.where(valid, s, -jnp.inf)

    sj3 = s[:, None, :]
    jl = lax.broadcasted_iota(jnp.int32, (rows, 16, W), 2)
    il0 = lax.broadcasted_iota(jnp.int32, (rows, 16, W), 1)
    rank_chunks = []
    for ic in range(W // 16):
        si3 = s[:, ic * 16:(ic + 1) * 16, None]
        beats = (sj3 > si3) | ((sj3 == si3) & (jl < il0 + ic * 16))
        cnt = jnp.sum(jnp.where(beats, 1.0, 0.0), axis=2)
        rank_chunks.append(cnt)
    rank = jnp.concatenate(rank_chunks, axis=1).astype(jnp.int32)

    pp = lax.broadcasted_iota(jnp.int32, (rows, TOPK, W), 1)
    sel = rank[:, None, :] == pp
    lane3 = lax.broadcasted_iota(jnp.int32, (rows, TOPK, W), 2)
    idx = jnp.sum(jnp.where(sel, lane3 - 1, 0), axis=2)

    idx_ref[...] = idx
    row = lax.broadcasted_iota(jnp.int32, (rows, TOPK), 0)
    gidx = idx + row * P
    gidx_ref[...] = jnp.concatenate([gidx, gidx], axis=1)


@functools.lru_cache(maxsize=None)
def _make_sc_gather(rows, d):
    n_rows = rows * TOPK
    per_w = n_rows // SC_WORKERS
    mesh = plsc.VectorSubcoreMesh(core_axis_name="c", subcore_axis_name="s")

    @functools.partial(
        pl.kernel,
        mesh=mesh,
        out_type=jax.ShapeDtypeStruct((n_rows, d), jnp.float32),
        scratch_types=[
            pltpu.VMEM((per_w // 2,), jnp.int32),
            pltpu.VMEM((per_w // 2,), jnp.int32),
            pltpu.VMEM((per_w, d), jnp.float32),
            pltpu.SemaphoreType.DMA,
            pltpu.SemaphoreType.DMA,
        ],
    )
    def sc_gather(table_hbm, gidx_hbm, out_hbm, idx_va, idx_vb, rows_v,
                  semg, semg2):
        wid = lax.axis_index("s") * SC_CORES + lax.axis_index("c")
        r = wid // 2
        c = wid % 2
        base = r * TOPK + c * per_w
        half = per_w // 2
        pltpu.sync_copy(gidx_hbm.at[r, pl.ds(c * per_w, half)], idx_va)
        pltpu.sync_copy(gidx_hbm.at[r, pl.ds(c * per_w + half, half)], idx_vb)
        h1 = pltpu.async_copy(
            table_hbm.at[idx_va], rows_v.at[pl.ds(0, half)], semg
        )
        h2 = pltpu.async_copy(
            table_hbm.at[idx_vb], rows_v.at[pl.ds(half, half)], semg2
        )
        h1.wait()
        pltpu.sync_copy(rows_v.at[pl.ds(0, half)], out_hbm.at[pl.ds(base, half)])
        h2.wait()
        pltpu.sync_copy(
            rows_v.at[pl.ds(half, half)], out_hbm.at[pl.ds(base + half, half)]
        )

    return sc_gather


def kernel(tokens, attn_maps):
    B = tokens.shape[0]
    rows = B * NUM_FRAME

    am_t = jnp.transpose(attn_maps, (0, 2, 3, 4, 1, 5))
    parts = _make_sc_scores(B)(am_t)

    idx, gidx = pl.pallas_call(
        functools.partial(_topk_body, batch=B),
        grid=(1,),
        in_specs=[
            pl.BlockSpec((B, NL * HG, NUM_FRAME, W), lambda i: (0, 0, 0, 0))
        ],
        out_specs=[
            pl.BlockSpec((rows, TOPK), lambda i: (0, 0)),
            pl.BlockSpec((rows, 2 * TOPK), lambda i: (0, 0)),
        ],
        out_shape=[
            jax.ShapeDtypeStruct((rows, TOPK), jnp.int32),
            jax.ShapeDtypeStruct((rows, 2 * TOPK), jnp.int32),
        ],
    )(parts.reshape(B, NL * HG, NUM_FRAME, W))

    gather = _make_sc_gather(rows, D)
    out = gather(tokens.reshape(B * NUM_FRAME * P, D), gidx)

    return out.reshape(B, NUM_FRAME * TOPK, D), idx.reshape(B, NUM_FRAME, TOPK)
